# Initial kernel scaffold; baseline (speedup 1.0000x reference)
import jax
import jax.numpy as jnp
from jax.experimental import pallas as pl


def kernel(x, W):
    raise NotImplementedError("write your pallas kernel here")


# speedup vs baseline: 3.5098x; 3.5098x over previous
import jax
import jax.numpy as jnp
from jax import lax
from jax.experimental import pallas as pl
from jax.experimental.pallas import tpu as pltpu
from jax.experimental.pallas import tpu_sc as plsc

_NC = 2
_NS = 16
_NW = _NC * _NS
_CH = 128


def kernel(x, W):
    batch, fields = x.shape
    n = batch * fields
    d = W.shape[1]
    idx = x.reshape(n)
    Wp = jnp.pad(W, ((0, 0), (0, 128 - d)))
    b_per_w = n // _NW
    ng = b_per_w // _CH
    mesh = plsc.VectorSubcoreMesh(core_axis_name="c", subcore_axis_name="s")

    @pl.kernel(
        out_type=jax.ShapeDtypeStruct((n, 128), W.dtype),
        mesh=mesh,
        scratch_types=[
            pltpu.VMEM((b_per_w,), jnp.int32),
            pltpu.VMEM((_CH, 128), W.dtype),
            pltpu.VMEM((_CH, 128), W.dtype),
            pltpu.SemaphoreType.DMA,
            pltpu.SemaphoreType.DMA,
        ],
    )
    def gather_kernel(w_hbm, i_hbm, o_hbm, idx_v, buf0, buf1, sem0, sem1):
        wid = lax.axis_index("s") * _NC + lax.axis_index("c")
        base = wid * b_per_w
        pltpu.sync_copy(i_hbm.at[pl.ds(base, b_per_w)], idx_v)

        def start(g, buf, sem):
            pltpu.async_copy(
                w_hbm.at[idx_v.at[pl.ds(g * _CH, _CH)]], buf, sem
            )

        def wait(buf, sem):
            pltpu.make_async_copy(
                w_hbm.at[pl.ds(0, _CH)], buf, sem
            ).wait()

        def writeout(g, buf):
            pltpu.sync_copy(buf, o_hbm.at[pl.ds(base + g * _CH, _CH)])

        start(0, buf0, sem0)
        start(1, buf1, sem1)

        @pl.loop(0, ng, step=2)
        def _(g):
            wait(buf0, sem0)
            writeout(g, buf0)

            @pl.when(g + 2 < ng)
            def _():
                start(g + 2, buf0, sem0)

            wait(buf1, sem1)
            writeout(g + 1, buf1)

            @pl.when(g + 3 < ng)
            def _():
                start(g + 3, buf1, sem1)

    out = gather_kernel(Wp, idx)
    return out[:, :d].reshape(batch, fields, d)
